# Initial kernel scaffold; baseline (speedup 1.0000x reference)
import jax
import jax.numpy as jnp
from jax.experimental import pallas as pl


def kernel(x, W0, W1, W2, W3):
    raise NotImplementedError("write your pallas kernel here")


# speedup vs baseline: 5.5229x; 5.5229x over previous
import functools

import jax
import jax.numpy as jnp
from jax import lax
from jax.experimental import pallas as pl
from jax.experimental.pallas import tpu as pltpu
from jax.experimental.pallas import tpu_sc as plsc

B = 4096
DENSE = 64
HIST = 50
N_EMB = 4
EMB_DIM = 64
XCOLS = DENSE + N_EMB * HIST
OCOLS = DENSE + N_EMB * EMB_DIM

NC, NS = 2, 16
NW = NC * NS
ROWS_PER_W = B // NW
RCHUNK = 8
NCHUNK = ROWS_PER_W // RCHUNK


def _body(x_hbm, w0, w1, w2, w3, out_hbm, xv, idxv, gbuf, ov, sem):
    tables = [w0, w1, w2, w3]
    wid = lax.axis_index("c") * NS + lax.axis_index("s")

    def chunk_body(chunk, _):
        base = wid * ROWS_PER_W + chunk * RCHUNK
        pltpu.sync_copy(x_hbm.at[pl.ds(base, RCHUNK), :], xv)

        for i in range(RCHUNK):
            for c in range(DENSE // 16):
                ov[i, pl.ds(16 * c, 16)] = xv[i, pl.ds(16 * c, 16)]
            for t in range(N_EMB):
                for k, off in enumerate((0, 16, 32, HIST - 16)):
                    src = DENSE + t * HIST + off
                    idxv[i * N_EMB + t, 0, pl.ds(off, 16)] = xv[
                        i, pl.ds(src, 16)
                    ].astype(jnp.int32)

        for i in range(RCHUNK):
            for t in range(N_EMB):
                idx_slice = idxv.at[i * N_EMB + t, 0]
                pltpu.async_copy(tables[t].at[idx_slice], gbuf, sem).wait()

                def red(j, carry):
                    s0, s1, s2, s3, cnt = carry
                    g0 = gbuf[j, pl.ds(0, 16)]
                    g1 = gbuf[j, pl.ds(16, 16)]
                    g2 = gbuf[j, pl.ds(32, 16)]
                    g3 = gbuf[j, pl.ds(48, 16)]
                    m = (g0 != 0.0) & (g1 != 0.0) & (g2 != 0.0) & (g3 != 0.0)
                    cnt = cnt + jnp.where(jnp.all(m), 1.0, 0.0)
                    return (s0 + g0, s1 + g1, s2 + g2, s3 + g3, cnt)

                zero = jnp.zeros((16,), jnp.float32)
                s0, s1, s2, s3, cnt = lax.fori_loop(
                    0, HIST, red, (zero, zero, zero, zero, zero)
                )
                div = jnp.where(cnt == 0.0, jnp.float32(1e-8), cnt)
                col = DENSE + EMB_DIM * t
                ov[i, pl.ds(col, 16)] = s0 / div
                ov[i, pl.ds(col + 16, 16)] = s1 / div
                ov[i, pl.ds(col + 32, 16)] = s2 / div
                ov[i, pl.ds(col + 48, 16)] = s3 / div

        pltpu.sync_copy(ov, out_hbm.at[pl.ds(base, RCHUNK), :])
        return ()

    lax.fori_loop(0, NCHUNK, chunk_body, ())


@jax.jit
def kernel(x, W0, W1, W2, W3):
    mesh = plsc.VectorSubcoreMesh(core_axis_name="c", subcore_axis_name="s")
    f = pl.kernel(
        _body,
        out_type=jax.ShapeDtypeStruct((B, OCOLS), jnp.float32),
        mesh=mesh,
        compiler_params=pltpu.CompilerParams(
            needs_layout_passes=False, use_tc_tiling_on_sc=False
        ),
        scratch_types=[
            pltpu.VMEM((RCHUNK, XCOLS), jnp.float32),
            pltpu.VMEM((RCHUNK * N_EMB, 1, HIST), jnp.int32),
            pltpu.VMEM((HIST, EMB_DIM), jnp.float32),
            pltpu.VMEM((RCHUNK, OCOLS), jnp.float32),
            pltpu.SemaphoreType.DMA,
        ],
    )
    return f(x, W0, W1, W2, W3)


# speedup vs baseline: 9.1762x; 1.6615x over previous
import jax
import jax.numpy as jnp
from jax import lax
from jax.experimental import pallas as pl
from jax.experimental.pallas import tpu as pltpu
from jax.experimental.pallas import tpu_sc as plsc

B = 4096
DENSE = 64
HIST = 50
N_EMB = 4
EMB_DIM = 64
XCOLS = DENSE + N_EMB * HIST
OCOLS = DENSE + N_EMB * EMB_DIM

NC, NS = 2, 16
NW = NC * NS
ROWS_PER_W = B // NW
RCHUNK = 8
NCHUNK = ROWS_PER_W // RCHUNK
NPAIR = RCHUNK // 2
NUNIT = NPAIR * N_EMB


def _body(x_hbm, w0, w1, w2, w3, out_hbm, xv, idxv, gb0, gb1, ov, s0, s1):
    tables = [w0, w1, w2, w3]
    gbufs = (gb0, gb1)
    sems = (s0, s1)
    wid = lax.axis_index("c") * NS + lax.axis_index("s")

    def reduce_rows(gbuf, row0, i, t):
        def red(j, carry):
            s0_, s1_, s2_, s3_, cnt = carry
            for jj in (2 * j, 2 * j + 1):
                g0 = gbuf[row0 + jj, pl.ds(0, 16)]
                g1 = gbuf[row0 + jj, pl.ds(16, 16)]
                g2 = gbuf[row0 + jj, pl.ds(32, 16)]
                g3 = gbuf[row0 + jj, pl.ds(48, 16)]
                m = (g0 != 0.0) & (g1 != 0.0) & (g2 != 0.0) & (g3 != 0.0)
                cnt = cnt + jnp.where(jnp.all(m), 1.0, 0.0)
                s0_, s1_, s2_, s3_ = s0_ + g0, s1_ + g1, s2_ + g2, s3_ + g3
            return (s0_, s1_, s2_, s3_, cnt)

        zero = jnp.zeros((16,), jnp.float32)
        s0_, s1_, s2_, s3_, cnt = lax.fori_loop(
            0, HIST // 2, red, (zero, zero, zero, zero, zero)
        )
        div = jnp.where(cnt == 0.0, jnp.float32(1e-8), cnt)
        col = DENSE + EMB_DIM * t
        ov[i, pl.ds(col, 16)] = s0_ / div
        ov[i, pl.ds(col + 16, 16)] = s1_ / div
        ov[i, pl.ds(col + 32, 16)] = s2_ / div
        ov[i, pl.ds(col + 48, 16)] = s3_ / div

    def chunk_body(chunk, _):
        base = wid * ROWS_PER_W + chunk * RCHUNK
        pltpu.sync_copy(x_hbm.at[pl.ds(base, RCHUNK), :], xv)

        for i in range(RCHUNK):
            for c in range(DENSE // 16):
                ov[i, pl.ds(16 * c, 16)] = xv[i, pl.ds(16 * c, 16)]
            for t in range(N_EMB):
                m = (i // 2) * N_EMB + t
                half = (i % 2) * HIST
                for off in (0, 16, 32, HIST - 16):
                    src = DENSE + t * HIST + off
                    idxv[m, 0, pl.ds(half + off, 16)] = xv[
                        i, pl.ds(src, 16)
                    ].astype(jnp.int32)

        units = [(p, t) for p in range(NPAIR) for t in range(N_EMB)]

        def issue(k):
            p, t = units[k]
            return pltpu.async_copy(
                tables[t].at[idxv.at[p * N_EMB + t, 0]], gbufs[k % 2], sems[k % 2]
            )

        handles = {0: issue(0)}
        for k in range(NUNIT):
            if k + 1 < NUNIT:
                handles[k + 1] = issue(k + 1)
            handles[k].wait()
            p, t = units[k]
            reduce_rows(gbufs[k % 2], 0, 2 * p, t)
            reduce_rows(gbufs[k % 2], HIST, 2 * p + 1, t)

        pltpu.sync_copy(ov, out_hbm.at[pl.ds(base, RCHUNK), :])
        return ()

    lax.fori_loop(0, NCHUNK, chunk_body, ())


@jax.jit
def kernel(x, W0, W1, W2, W3):
    mesh = plsc.VectorSubcoreMesh(core_axis_name="c", subcore_axis_name="s")
    f = pl.kernel(
        _body,
        out_type=jax.ShapeDtypeStruct((B, OCOLS), jnp.float32),
        mesh=mesh,
        compiler_params=pltpu.CompilerParams(
            needs_layout_passes=False, use_tc_tiling_on_sc=False
        ),
        scratch_types=[
            pltpu.VMEM((RCHUNK, XCOLS), jnp.float32),
            pltpu.VMEM((NUNIT, 1, 2 * HIST), jnp.int32),
            pltpu.VMEM((2 * HIST, EMB_DIM), jnp.float32),
            pltpu.VMEM((2 * HIST, EMB_DIM), jnp.float32),
            pltpu.VMEM((RCHUNK, OCOLS), jnp.float32),
            pltpu.SemaphoreType.DMA,
            pltpu.SemaphoreType.DMA,
        ],
    )
    return f(x, W0, W1, W2, W3)


# speedup vs baseline: 9.6844x; 1.0554x over previous
import jax
import jax.numpy as jnp
from jax import lax
from jax.experimental import pallas as pl
from jax.experimental.pallas import tpu as pltpu
from jax.experimental.pallas import tpu_sc as plsc

B = 4096
DENSE = 64
HIST = 50
N_EMB = 4
EMB_DIM = 64
XCOLS = DENSE + N_EMB * HIST
OCOLS = DENSE + N_EMB * EMB_DIM

NC, NS = 2, 16
NW = NC * NS
ROWS_PER_W = B // NW
RCHUNK = 8
NCHUNK = ROWS_PER_W // RCHUNK
NPAIR = RCHUNK // 2
NUNIT = NPAIR * N_EMB


def _body(x_hbm, w0, w1, w2, w3, out_hbm, xv, idxv, gb0, gb1, ov, s0, s1):
    tables = [w0, w1, w2, w3]
    gbufs = (gb0, gb1)
    sems = (s0, s1)
    wid = lax.axis_index("c") * NS + lax.axis_index("s")

    def reduce_rows(gbuf, row0, i, t):
        def red(j, carry):
            s0_, s1_, s2_, s3_, cnt = carry
            for jj in (2 * j, 2 * j + 1):
                g0 = gbuf[row0 + jj, pl.ds(0, 16)]
                g1 = gbuf[row0 + jj, pl.ds(16, 16)]
                g2 = gbuf[row0 + jj, pl.ds(32, 16)]
                g3 = gbuf[row0 + jj, pl.ds(48, 16)]
                mag = jnp.float32(0)
                for g in (g0, g1, g2, g3):
                    a = plsc.bitcast(g, jnp.int32) & jnp.int32(0x7FFFFFFF)
                    mag = a if g is g0 else jnp.minimum(mag, a)
                pcnt = plsc.all_reduce_population_count(mag > 0)
                cnt = cnt + jnp.where(pcnt == 16, 1.0, 0.0)
                s0_, s1_, s2_, s3_ = s0_ + g0, s1_ + g1, s2_ + g2, s3_ + g3
            return (s0_, s1_, s2_, s3_, cnt)

        zero = jnp.zeros((16,), jnp.float32)
        s0_, s1_, s2_, s3_, cnt = lax.fori_loop(
            0, HIST // 2, red, (zero, zero, zero, zero, zero)
        )
        div = jnp.where(cnt == 0.0, jnp.float32(1e-8), cnt)
        col = DENSE + EMB_DIM * t
        ov[i, pl.ds(col, 16)] = s0_ / div
        ov[i, pl.ds(col + 16, 16)] = s1_ / div
        ov[i, pl.ds(col + 32, 16)] = s2_ / div
        ov[i, pl.ds(col + 48, 16)] = s3_ / div

    def chunk_body(chunk, _):
        base = wid * ROWS_PER_W + chunk * RCHUNK
        pltpu.sync_copy(x_hbm.at[pl.ds(base, RCHUNK), :], xv)

        for i in range(RCHUNK):
            for c in range(DENSE // 16):
                ov[i, pl.ds(16 * c, 16)] = xv[i, pl.ds(16 * c, 16)]
            for t in range(N_EMB):
                m = (i // 2) * N_EMB + t
                half = (i % 2) * HIST
                for off in (0, 16, 32, HIST - 16):
                    src = DENSE + t * HIST + off
                    idxv[m, 0, pl.ds(half + off, 16)] = xv[
                        i, pl.ds(src, 16)
                    ].astype(jnp.int32)

        units = [(p, t) for p in range(NPAIR) for t in range(N_EMB)]

        def issue(k):
            p, t = units[k]
            return pltpu.async_copy(
                tables[t].at[idxv.at[p * N_EMB + t, 0]], gbufs[k % 2], sems[k % 2]
            )

        handles = {0: issue(0)}
        for k in range(NUNIT):
            if k + 1 < NUNIT:
                handles[k + 1] = issue(k + 1)
            handles[k].wait()
            p, t = units[k]
            reduce_rows(gbufs[k % 2], 0, 2 * p, t)
            reduce_rows(gbufs[k % 2], HIST, 2 * p + 1, t)

        pltpu.sync_copy(ov, out_hbm.at[pl.ds(base, RCHUNK), :])
        return ()

    lax.fori_loop(0, NCHUNK, chunk_body, ())


@jax.jit
def kernel(x, W0, W1, W2, W3):
    mesh = plsc.VectorSubcoreMesh(core_axis_name="c", subcore_axis_name="s")
    f = pl.kernel(
        _body,
        out_type=jax.ShapeDtypeStruct((B, OCOLS), jnp.float32),
        mesh=mesh,
        compiler_params=pltpu.CompilerParams(
            needs_layout_passes=False, use_tc_tiling_on_sc=False
        ),
        scratch_types=[
            pltpu.VMEM((RCHUNK, XCOLS), jnp.float32),
            pltpu.VMEM((NUNIT, 1, 2 * HIST), jnp.int32),
            pltpu.VMEM((2 * HIST, EMB_DIM), jnp.float32),
            pltpu.VMEM((2 * HIST, EMB_DIM), jnp.float32),
            pltpu.VMEM((RCHUNK, OCOLS), jnp.float32),
            pltpu.SemaphoreType.DMA,
            pltpu.SemaphoreType.DMA,
        ],
    )
    return f(x, W0, W1, W2, W3)


# speedup vs baseline: 11.1199x; 1.1482x over previous
import jax
import jax.numpy as jnp
from jax import lax
from jax.experimental import pallas as pl
from jax.experimental.pallas import tpu as pltpu
from jax.experimental.pallas import tpu_sc as plsc

B = 4096
DENSE = 64
HIST = 50
N_EMB = 4
EMB_DIM = 64
XCOLS = DENSE + N_EMB * HIST
OCOLS = DENSE + N_EMB * EMB_DIM

NC, NS = 2, 16
NW = NC * NS
ROWS_PER_W = B // NW
RCHUNK = 16
NCHUNK = ROWS_PER_W // RCHUNK
NPAIR = RCHUNK // 2
NUNIT = NPAIR * N_EMB
NBUF = 3


def _body(x_hbm, w0, w1, w2, w3, out_hbm, xv, idxv, gb0, gb1, gb2, ov, s0, s1, s2):
    tables = [w0, w1, w2, w3]
    gbufs = (gb0, gb1, gb2)
    sems = (s0, s1, s2)
    wid = lax.axis_index("c") * NS + lax.axis_index("s")

    def row_step(gbuf, row, sums, cnt):
        g0 = gbuf[row, pl.ds(0, 16)]
        g1 = gbuf[row, pl.ds(16, 16)]
        g2 = gbuf[row, pl.ds(32, 16)]
        g3 = gbuf[row, pl.ds(48, 16)]
        mag = jnp.float32(0)
        for g in (g0, g1, g2, g3):
            a = plsc.bitcast(g, jnp.int32) & jnp.int32(0x7FFFFFFF)
            mag = a if g is g0 else jnp.minimum(mag, a)
        pcnt = plsc.all_reduce_population_count(mag > 0)
        cnt = cnt + jnp.where(pcnt == 16, 1.0, 0.0)
        return (sums[0] + g0, sums[1] + g1, sums[2] + g2, sums[3] + g3), cnt

    def write_row(i, t, sums, cnt):
        div = jnp.where(cnt == 0.0, jnp.float32(1e-8), cnt)
        col = DENSE + EMB_DIM * t
        for c in range(4):
            ov[i, pl.ds(col + 16 * c, 16)] = sums[c] / div

    def reduce_pair(gbuf, p, t):
        def red(j, carry):
            sa, ca, sb, cb = carry
            for jj in (2 * j, 2 * j + 1):
                sa, ca = row_step(gbuf, jj, sa, ca)
                sb, cb = row_step(gbuf, HIST + jj, sb, cb)
            return (sa, ca, sb, cb)

        zero = jnp.zeros((16,), jnp.float32)
        z4 = (zero, zero, zero, zero)
        sa, ca, sb, cb = lax.fori_loop(0, HIST // 2, red, (z4, zero, z4, zero))
        write_row(2 * p, t, sa, ca)
        write_row(2 * p + 1, t, sb, cb)

    def chunk_body(chunk, _):
        base = wid * ROWS_PER_W + chunk * RCHUNK
        pltpu.sync_copy(x_hbm.at[pl.ds(base, RCHUNK), :], xv)

        for i in range(RCHUNK):
            for c in range(DENSE // 16):
                ov[i, pl.ds(16 * c, 16)] = xv[i, pl.ds(16 * c, 16)]
            for t in range(N_EMB):
                m = (i // 2) * N_EMB + t
                half = (i % 2) * HIST
                for off in (0, 16, 32, HIST - 16):
                    src = DENSE + t * HIST + off
                    idxv[m, 0, pl.ds(half + off, 16)] = xv[
                        i, pl.ds(src, 16)
                    ].astype(jnp.int32)

        units = [(p, t) for p in range(NPAIR) for t in range(N_EMB)]

        def issue(k):
            p, t = units[k]
            return pltpu.async_copy(
                tables[t].at[idxv.at[p * N_EMB + t, 0]],
                gbufs[k % NBUF],
                sems[k % NBUF],
            )

        handles = {k: issue(k) for k in range(NBUF - 1)}
        for k in range(NUNIT):
            if k + NBUF - 1 < NUNIT:
                handles[k + NBUF - 1] = issue(k + NBUF - 1)
            handles[k].wait()
            p, t = units[k]
            reduce_pair(gbufs[k % NBUF], p, t)

        pltpu.sync_copy(ov, out_hbm.at[pl.ds(base, RCHUNK), :])
        return ()

    lax.fori_loop(0, NCHUNK, chunk_body, ())


@jax.jit
def kernel(x, W0, W1, W2, W3):
    mesh = plsc.VectorSubcoreMesh(core_axis_name="c", subcore_axis_name="s")
    f = pl.kernel(
        _body,
        out_type=jax.ShapeDtypeStruct((B, OCOLS), jnp.float32),
        mesh=mesh,
        compiler_params=pltpu.CompilerParams(
            needs_layout_passes=False, use_tc_tiling_on_sc=False
        ),
        scratch_types=[
            pltpu.VMEM((RCHUNK, XCOLS), jnp.float32),
            pltpu.VMEM((NUNIT, 1, 2 * HIST), jnp.int32),
            pltpu.VMEM((2 * HIST, EMB_DIM), jnp.float32),
            pltpu.VMEM((2 * HIST, EMB_DIM), jnp.float32),
            pltpu.VMEM((2 * HIST, EMB_DIM), jnp.float32),
            pltpu.VMEM((RCHUNK, OCOLS), jnp.float32),
            pltpu.SemaphoreType.DMA,
            pltpu.SemaphoreType.DMA,
            pltpu.SemaphoreType.DMA,
        ],
    )
    return f(x, W0, W1, W2, W3)


# speedup vs baseline: 11.6983x; 1.0520x over previous
import jax
import jax.numpy as jnp
from jax import lax
from jax.experimental import pallas as pl
from jax.experimental.pallas import tpu as pltpu
from jax.experimental.pallas import tpu_sc as plsc

B = 4096
DENSE = 64
HIST = 50
N_EMB = 4
EMB_DIM = 64
XCOLS = DENSE + N_EMB * HIST

NC, NS = 2, 16
NW = NC * NS
ROWS_PER_W = B // NW
RCHUNK = 16
NCHUNK = ROWS_PER_W // RCHUNK
NPAIR = RCHUNK // 2
NBUF = 3


def _make_body(t):
    with_dense = t == 0
    ocols = DENSE + EMB_DIM if with_dense else EMB_DIM
    ecol = DENSE if with_dense else 0

    def body(x_hbm, w, out_hbm, xv, idxv, gb0, gb1, gb2, ov, s0, s1, s2):
        gbufs = (gb0, gb1, gb2)
        sems = (s0, s1, s2)
        wid = lax.axis_index("c") * NS + lax.axis_index("s")

        def row_step(gbuf, row, sums, cnt):
            g0 = gbuf[row, pl.ds(0, 16)]
            g1 = gbuf[row, pl.ds(16, 16)]
            g2 = gbuf[row, pl.ds(32, 16)]
            g3 = gbuf[row, pl.ds(48, 16)]
            mag = jnp.float32(0)
            for g in (g0, g1, g2, g3):
                a = plsc.bitcast(g, jnp.int32) & jnp.int32(0x7FFFFFFF)
                mag = a if g is g0 else jnp.minimum(mag, a)
            pcnt = plsc.all_reduce_population_count(mag > 0)
            cnt = cnt + jnp.where(pcnt == 16, 1.0, 0.0)
            return (sums[0] + g0, sums[1] + g1, sums[2] + g2, sums[3] + g3), cnt

        def write_row(i, sums, cnt):
            div = jnp.where(cnt == 0.0, jnp.float32(1e-8), cnt)
            for c in range(4):
                ov[i, pl.ds(ecol + 16 * c, 16)] = sums[c] / div

        def reduce_pair(gbuf, p):
            def red(j, carry):
                sa, ca, sb, cb = carry
                for jj in (2 * j, 2 * j + 1):
                    sa, ca = row_step(gbuf, jj, sa, ca)
                    sb, cb = row_step(gbuf, HIST + jj, sb, cb)
                return (sa, ca, sb, cb)

            zero = jnp.zeros((16,), jnp.float32)
            z4 = (zero, zero, zero, zero)
            sa, ca, sb, cb = lax.fori_loop(0, HIST // 2, red, (z4, zero, z4, zero))
            write_row(2 * p, sa, ca)
            write_row(2 * p + 1, sb, cb)

        def chunk_body(chunk, _):
            base = wid * ROWS_PER_W + chunk * RCHUNK
            pltpu.sync_copy(x_hbm.at[pl.ds(base, RCHUNK), :], xv)

            for i in range(RCHUNK):
                if with_dense:
                    for c in range(DENSE // 16):
                        ov[i, pl.ds(16 * c, 16)] = xv[i, pl.ds(16 * c, 16)]
                half = (i % 2) * HIST
                for off in (0, 16, 32, HIST - 16):
                    src = DENSE + t * HIST + off
                    idxv[i // 2, 0, pl.ds(half + off, 16)] = xv[
                        i, pl.ds(src, 16)
                    ].astype(jnp.int32)

            def issue(k):
                return pltpu.async_copy(
                    w.at[idxv.at[k, 0]], gbufs[k % NBUF], sems[k % NBUF]
                )

            handles = {k: issue(k) for k in range(NBUF - 1)}
            for k in range(NPAIR):
                if k + NBUF - 1 < NPAIR:
                    handles[k + NBUF - 1] = issue(k + NBUF - 1)
                handles[k].wait()
                reduce_pair(gbufs[k % NBUF], k)

            pltpu.sync_copy(ov, out_hbm.at[pl.ds(base, RCHUNK), :])
            return ()

        lax.fori_loop(0, NCHUNK, chunk_body, ())

    mesh = plsc.VectorSubcoreMesh(core_axis_name="c", subcore_axis_name="s")
    return pl.kernel(
        body,
        out_type=jax.ShapeDtypeStruct((B, ocols), jnp.float32),
        mesh=mesh,
        compiler_params=pltpu.CompilerParams(
            needs_layout_passes=False, use_tc_tiling_on_sc=False
        ),
        scratch_types=[
            pltpu.VMEM((RCHUNK, XCOLS), jnp.float32),
            pltpu.VMEM((NPAIR, 1, 2 * HIST), jnp.int32),
            pltpu.VMEM((2 * HIST, EMB_DIM), jnp.float32),
            pltpu.VMEM((2 * HIST, EMB_DIM), jnp.float32),
            pltpu.VMEM((2 * HIST, EMB_DIM), jnp.float32),
            pltpu.VMEM((RCHUNK, ocols), jnp.float32),
            pltpu.SemaphoreType.DMA,
            pltpu.SemaphoreType.DMA,
            pltpu.SemaphoreType.DMA,
        ],
        name=f"emb_pool_t{t}",
    )


_CALLS = [_make_body(t) for t in range(N_EMB)]


@jax.jit
def kernel(x, W0, W1, W2, W3):
    parts = [f(x, w) for f, w in zip(_CALLS, (W0, W1, W2, W3))]
    return jnp.concatenate(parts, axis=1)


# speedup vs baseline: 12.4901x; 1.0677x over previous
import jax
import jax.numpy as jnp
from jax import lax
from jax.experimental import pallas as pl
from jax.experimental.pallas import tpu as pltpu
from jax.experimental.pallas import tpu_sc as plsc

B = 4096
DENSE = 64
HIST = 50
N_EMB = 4
EMB_DIM = 64
XCOLS = DENSE + N_EMB * HIST

NC, NS = 2, 16
NW = NC * NS
ROWS_PER_W = B // NW
RCHUNK = 32
NCHUNK = ROWS_PER_W // RCHUNK
NPAIR = RCHUNK // 2
NBUF = 3


def _make_body(t):
    with_dense = t == 0
    ocols = DENSE + EMB_DIM if with_dense else EMB_DIM
    ecol = DENSE if with_dense else 0
    raw0 = 0 if with_dense else DENSE + t * HIST
    xcol0 = (raw0 // 8) * 8
    icol = (DENSE if with_dense else 0) + (raw0 - xcol0)
    xcols = ((icol + HIST + 7) // 8) * 8

    def body(x_hbm, w, out_hbm, xv, idxv, gb0, gb1, gb2, ov, s0, s1, s2):
        gbufs = (gb0, gb1, gb2)
        sems = (s0, s1, s2)
        wid = lax.axis_index("c") * NS + lax.axis_index("s")

        def row_step(gbuf, row, sums, cnt):
            g0 = gbuf[row, pl.ds(0, 16)]
            g1 = gbuf[row, pl.ds(16, 16)]
            g2 = gbuf[row, pl.ds(32, 16)]
            g3 = gbuf[row, pl.ds(48, 16)]
            mag = jnp.float32(0)
            for g in (g0, g1, g2, g3):
                a = plsc.bitcast(g, jnp.int32) & jnp.int32(0x7FFFFFFF)
                mag = a if g is g0 else jnp.minimum(mag, a)
            pcnt = plsc.all_reduce_population_count(mag > 0)
            cnt = cnt + jnp.where(pcnt == 16, 1.0, 0.0)
            return (sums[0] + g0, sums[1] + g1, sums[2] + g2, sums[3] + g3), cnt

        def write_row(i, sums, cnt):
            div = jnp.where(cnt == 0.0, jnp.float32(1e-8), cnt)
            for c in range(4):
                ov[i, pl.ds(ecol + 16 * c, 16)] = sums[c] / div

        def reduce_pair(gbuf, p):
            def red(j, carry):
                sa, ca, sb, cb = carry
                for jj in (2 * j, 2 * j + 1):
                    sa, ca = row_step(gbuf, jj, sa, ca)
                    sb, cb = row_step(gbuf, HIST + jj, sb, cb)
                return (sa, ca, sb, cb)

            zero = jnp.zeros((16,), jnp.float32)
            z4 = (zero, zero, zero, zero)
            sa, ca, sb, cb = lax.fori_loop(0, HIST // 2, red, (z4, zero, z4, zero))
            write_row(2 * p, sa, ca)
            write_row(2 * p + 1, sb, cb)

        def chunk_body(chunk, _):
            base = wid * ROWS_PER_W + chunk * RCHUNK
            pltpu.sync_copy(
                x_hbm.at[pl.ds(base, RCHUNK), pl.ds(xcol0, xcols)], xv
            )

            for i in range(RCHUNK):
                if with_dense:
                    for c in range(DENSE // 16):
                        ov[i, pl.ds(16 * c, 16)] = xv[i, pl.ds(16 * c, 16)]
                half = (i % 2) * HIST
                for off in (0, 16, 32, HIST - 16):
                    idxv[i // 2, 0, pl.ds(half + off, 16)] = xv[
                        i, pl.ds(icol + off, 16)
                    ].astype(jnp.int32)

            def issue(k):
                return pltpu.async_copy(
                    w.at[idxv.at[k, 0]], gbufs[k % NBUF], sems[k % NBUF]
                )

            handles = {k: issue(k) for k in range(NBUF - 1)}
            for k in range(NPAIR):
                if k + NBUF - 1 < NPAIR:
                    handles[k + NBUF - 1] = issue(k + NBUF - 1)
                handles[k].wait()
                reduce_pair(gbufs[k % NBUF], k)

            pltpu.sync_copy(ov, out_hbm.at[pl.ds(base, RCHUNK), :])
            return ()

        lax.fori_loop(0, NCHUNK, chunk_body, ())

    mesh = plsc.VectorSubcoreMesh(core_axis_name="c", subcore_axis_name="s")
    return pl.kernel(
        body,
        out_type=jax.ShapeDtypeStruct((B, ocols), jnp.float32),
        mesh=mesh,
        compiler_params=pltpu.CompilerParams(
            needs_layout_passes=False, use_tc_tiling_on_sc=False
        ),
        scratch_types=[
            pltpu.VMEM((RCHUNK, xcols), jnp.float32),
            pltpu.VMEM((NPAIR, 1, 2 * HIST), jnp.int32),
            pltpu.VMEM((2 * HIST, EMB_DIM), jnp.float32),
            pltpu.VMEM((2 * HIST, EMB_DIM), jnp.float32),
            pltpu.VMEM((2 * HIST, EMB_DIM), jnp.float32),
            pltpu.VMEM((RCHUNK, ocols), jnp.float32),
            pltpu.SemaphoreType.DMA,
            pltpu.SemaphoreType.DMA,
            pltpu.SemaphoreType.DMA,
        ],
        name=f"emb_pool_t{t}",
    )


_CALLS = [_make_body(t) for t in range(N_EMB)]


@jax.jit
def kernel(x, W0, W1, W2, W3):
    parts = [f(x, w) for f, w in zip(_CALLS, (W0, W1, W2, W3))]
    return jnp.concatenate(parts, axis=1)


# speedup vs baseline: 12.5715x; 1.0065x over previous
import jax
import jax.numpy as jnp
from jax import lax
from jax.experimental import pallas as pl
from jax.experimental.pallas import tpu as pltpu
from jax.experimental.pallas import tpu_sc as plsc

B = 4096
DENSE = 64
HIST = 50
N_EMB = 4
EMB_DIM = 64
XCOLS = DENSE + N_EMB * HIST

NC, NS = 2, 16
NW = NC * NS
ROWS_PER_W = B // NW
RCHUNK = 64
NCHUNK = ROWS_PER_W // RCHUNK
NPAIR = RCHUNK // 2
NBUF = 3


def _make_body(t):
    with_dense = t == 0
    ocols = DENSE + EMB_DIM if with_dense else EMB_DIM
    ecol = DENSE if with_dense else 0
    raw0 = 0 if with_dense else DENSE + t * HIST
    xcol0 = (raw0 // 8) * 8
    icol = (DENSE if with_dense else 0) + (raw0 - xcol0)
    xcols = ((icol + HIST + 7) // 8) * 8

    def body(x_hbm, w, out_hbm, xv, idxv, gb0, gb1, gb2, ov, s0, s1, s2):
        gbufs = (gb0, gb1, gb2)
        sems = (s0, s1, s2)
        wid = lax.axis_index("c") * NS + lax.axis_index("s")

        def row_step(gbuf, row, sums, cnt):
            g0 = gbuf[row, pl.ds(0, 16)]
            g1 = gbuf[row, pl.ds(16, 16)]
            g2 = gbuf[row, pl.ds(32, 16)]
            g3 = gbuf[row, pl.ds(48, 16)]
            mag = jnp.float32(0)
            for g in (g0, g1, g2, g3):
                a = plsc.bitcast(g, jnp.int32) & jnp.int32(0x7FFFFFFF)
                mag = a if g is g0 else jnp.minimum(mag, a)
            pcnt = plsc.all_reduce_population_count(mag > 0)
            cnt = cnt + jnp.where(pcnt == 16, 1.0, 0.0)
            return (sums[0] + g0, sums[1] + g1, sums[2] + g2, sums[3] + g3), cnt

        def write_row(i, sums, cnt):
            div = jnp.where(cnt == 0.0, jnp.float32(1e-8), cnt)
            for c in range(4):
                ov[i, pl.ds(ecol + 16 * c, 16)] = sums[c] / div

        def reduce_pair(gbuf, p):
            def red(j, carry):
                sa, ca, sb, cb = carry
                for jj in (2 * j, 2 * j + 1):
                    sa, ca = row_step(gbuf, jj, sa, ca)
                    sb, cb = row_step(gbuf, HIST + jj, sb, cb)
                return (sa, ca, sb, cb)

            zero = jnp.zeros((16,), jnp.float32)
            z4 = (zero, zero, zero, zero)
            sa, ca, sb, cb = lax.fori_loop(0, HIST // 2, red, (z4, zero, z4, zero))
            write_row(2 * p, sa, ca)
            write_row(2 * p + 1, sb, cb)

        def chunk_body(chunk, _):
            base = wid * ROWS_PER_W + chunk * RCHUNK
            pltpu.sync_copy(
                x_hbm.at[pl.ds(base, RCHUNK), pl.ds(xcol0, xcols)], xv
            )

            for i in range(RCHUNK):
                if with_dense:
                    for c in range(DENSE // 16):
                        ov[i, pl.ds(16 * c, 16)] = xv[i, pl.ds(16 * c, 16)]
                half = (i % 2) * HIST
                for off in (0, 16, 32, HIST - 16):
                    idxv[i // 2, 0, pl.ds(half + off, 16)] = xv[
                        i, pl.ds(icol + off, 16)
                    ].astype(jnp.int32)

            def issue(k):
                return pltpu.async_copy(
                    w.at[idxv.at[k, 0]], gbufs[k % NBUF], sems[k % NBUF]
                )

            handles = {k: issue(k) for k in range(NBUF - 1)}
            for k in range(NPAIR):
                if k + NBUF - 1 < NPAIR:
                    handles[k + NBUF - 1] = issue(k + NBUF - 1)
                handles[k].wait()
                reduce_pair(gbufs[k % NBUF], k)

            pltpu.sync_copy(ov, out_hbm.at[pl.ds(base, RCHUNK), :])
            return ()

        lax.fori_loop(0, NCHUNK, chunk_body, ())

    mesh = plsc.VectorSubcoreMesh(core_axis_name="c", subcore_axis_name="s")
    return pl.kernel(
        body,
        out_type=jax.ShapeDtypeStruct((B, ocols), jnp.float32),
        mesh=mesh,
        compiler_params=pltpu.CompilerParams(
            needs_layout_passes=False, use_tc_tiling_on_sc=False
        ),
        scratch_types=[
            pltpu.VMEM((RCHUNK, xcols), jnp.float32),
            pltpu.VMEM((NPAIR, 1, 2 * HIST), jnp.int32),
            pltpu.VMEM((2 * HIST, EMB_DIM), jnp.float32),
            pltpu.VMEM((2 * HIST, EMB_DIM), jnp.float32),
            pltpu.VMEM((2 * HIST, EMB_DIM), jnp.float32),
            pltpu.VMEM((RCHUNK, ocols), jnp.float32),
            pltpu.SemaphoreType.DMA,
            pltpu.SemaphoreType.DMA,
            pltpu.SemaphoreType.DMA,
        ],
        name=f"emb_pool_t{t}",
    )


_CALLS = [_make_body(t) for t in range(N_EMB)]


@jax.jit
def kernel(x, W0, W1, W2, W3):
    parts = [f(x, w) for f, w in zip(_CALLS, (W0, W1, W2, W3))]
    return jnp.concatenate(parts, axis=1)


# speedup vs baseline: 12.5872x; 1.0012x over previous
import jax
import jax.numpy as jnp
from jax import lax
from jax.experimental import pallas as pl
from jax.experimental.pallas import tpu as pltpu
from jax.experimental.pallas import tpu_sc as plsc

B = 4096
DENSE = 64
HIST = 50
N_EMB = 4
EMB_DIM = 64
XCOLS = DENSE + N_EMB * HIST

NC, NS = 2, 16
NW = NC * NS
ROWS_PER_W = B // NW
RCHUNK = 64
NCHUNK = ROWS_PER_W // RCHUNK
NPAIR = RCHUNK // 2
NBUF = 3


def _make_body(t):
    with_dense = t == 0
    ocols = DENSE + EMB_DIM if with_dense else EMB_DIM
    ecol = DENSE if with_dense else 0
    raw0 = 0 if with_dense else DENSE + t * HIST
    xcol0 = (raw0 // 8) * 8
    icol = (DENSE if with_dense else 0) + (raw0 - xcol0)
    xcols = ((icol + HIST + 7) // 8) * 8

    def body(x_hbm, w, out_hbm, xv, idxv, gb0, gb1, gb2, ov, s0, s1, s2):
        gbufs = (gb0, gb1, gb2)
        sems = (s0, s1, s2)
        wid = lax.axis_index("c") * NS + lax.axis_index("s")

        def row_step(gbuf, row, sums, cnt):
            g0 = gbuf[row, pl.ds(0, 16)]
            g1 = gbuf[row, pl.ds(16, 16)]
            g2 = gbuf[row, pl.ds(32, 16)]
            g3 = gbuf[row, pl.ds(48, 16)]
            mag = jnp.float32(0)
            for g in (g0, g1, g2, g3):
                a = plsc.bitcast(g, jnp.int32) & jnp.int32(0x7FFFFFFF)
                mag = a if g is g0 else jnp.minimum(mag, a)
            pcnt = plsc.all_reduce_population_count(mag > 0)
            cnt = cnt + jnp.where(pcnt == 16, 1.0, 0.0)
            return (sums[0] + g0, sums[1] + g1, sums[2] + g2, sums[3] + g3), cnt

        def write_row(i, sums, cnt):
            div = jnp.where(cnt == 0.0, jnp.float32(1e-8), cnt)
            for c in range(4):
                ov[i, pl.ds(ecol + 16 * c, 16)] = sums[c] / div

        def reduce_pair(gbuf, p):
            def red(j, carry):
                sa, ca, sb, cb = carry
                for jj in (2 * j, 2 * j + 1):
                    sa, ca = row_step(gbuf, jj, sa, ca)
                    sb, cb = row_step(gbuf, HIST + jj, sb, cb)
                return (sa, ca, sb, cb)

            zero = jnp.zeros((16,), jnp.float32)
            z4 = (zero, zero, zero, zero)
            sa, ca, sb, cb = lax.fori_loop(0, HIST // 2, red, (z4, zero, z4, zero))
            write_row(2 * p, sa, ca)
            write_row(2 * p + 1, sb, cb)

        def chunk_body(chunk, _):
            base = wid * ROWS_PER_W + chunk * RCHUNK
            pltpu.sync_copy(
                x_hbm.at[pl.ds(base, RCHUNK), pl.ds(xcol0, xcols)], xv
            )

            for i in range(RCHUNK):
                if with_dense:
                    for c in range(DENSE // 16):
                        ov[i, pl.ds(16 * c, 16)] = xv[i, pl.ds(16 * c, 16)]
                half = (i % 2) * HIST
                for off in (0, 16, 32, HIST - 16):
                    idxv[i // 2, 0, pl.ds(half + off, 16)] = xv[
                        i, pl.ds(icol + off, 16)
                    ].astype(jnp.int32)

            def issue(k):
                return pltpu.async_copy(
                    w.at[idxv.at[k, 0]], gbufs[k % NBUF], sems[k % NBUF]
                )

            handles = {k: issue(k) for k in range(NBUF - 1)}
            for k in range(NPAIR):
                if k + NBUF - 1 < NPAIR:
                    handles[k + NBUF - 1] = issue(k + NBUF - 1)
                handles[k].wait()
                reduce_pair(gbufs[k % NBUF], k)

            pltpu.sync_copy(ov, out_hbm.at[pl.ds(base, RCHUNK), :])
            return ()

        lax.fori_loop(0, NCHUNK, chunk_body, ())

    mesh = plsc.VectorSubcoreMesh(core_axis_name="c", subcore_axis_name="s")
    return pl.kernel(
        body,
        out_type=jax.ShapeDtypeStruct((B, ocols), jnp.float32),
        mesh=mesh,
        compiler_params=pltpu.CompilerParams(
            needs_layout_passes=False, use_tc_tiling_on_sc=False
        ),
        scratch_types=[
            pltpu.VMEM((RCHUNK, xcols), jnp.float32),
            pltpu.VMEM((NPAIR, 1, 2 * HIST), jnp.int32),
            pltpu.VMEM((2 * HIST, EMB_DIM), jnp.float32),
            pltpu.VMEM((2 * HIST, EMB_DIM), jnp.float32),
            pltpu.VMEM((2 * HIST, EMB_DIM), jnp.float32),
            pltpu.VMEM((RCHUNK, ocols), jnp.float32),
            pltpu.SemaphoreType.DMA,
            pltpu.SemaphoreType.DMA,
            pltpu.SemaphoreType.DMA,
        ],
        name=f"emb_pool_t{t}",
    )


_CALLS = [_make_body(t) for t in range(N_EMB)]


@jax.jit
def kernel(x, W0, W1, W2, W3):
    ws = (W0, W1, W2, W3)
    parts = [None] * N_EMB
    for t in (1, 2, 3, 0):
        parts[t] = _CALLS[t](x, ws[t])
    return jnp.concatenate(parts, axis=1)


# speedup vs baseline: 13.0312x; 1.0353x over previous
import jax
import jax.numpy as jnp
from jax import lax
from jax.experimental import pallas as pl
from jax.experimental.pallas import tpu as pltpu
from jax.experimental.pallas import tpu_sc as plsc

B = 4096
DENSE = 64
HIST = 50
N_EMB = 4
EMB_DIM = 64
XCOLS = DENSE + N_EMB * HIST

NC, NS = 2, 16
NW = NC * NS
ROWS_PER_W = B // NW
RCHUNK = 64
NCHUNK = ROWS_PER_W // RCHUNK
NPAIR = RCHUNK // 2
NBUF = 4


def _make_body(t):
    with_dense = t == 0
    ocols = DENSE + EMB_DIM if with_dense else EMB_DIM
    ecol = DENSE if with_dense else 0
    raw0 = 0 if with_dense else DENSE + t * HIST
    xcol0 = (raw0 // 8) * 8
    icol = (DENSE if with_dense else 0) + (raw0 - xcol0)
    xcols = ((icol + HIST + 7) // 8) * 8

    def body(x_hbm, w, out_hbm, xv, idxv, gb0, gb1, gb2, gb3, ov, s0, s1, s2, s3):
        gbufs = (gb0, gb1, gb2, gb3)
        sems = (s0, s1, s2, s3)
        wid = lax.axis_index("c") * NS + lax.axis_index("s")

        def row_step(gbuf, row, sums, cnt):
            g0 = gbuf[row, pl.ds(0, 16)]
            g1 = gbuf[row, pl.ds(16, 16)]
            g2 = gbuf[row, pl.ds(32, 16)]
            g3 = gbuf[row, pl.ds(48, 16)]
            mag = jnp.float32(0)
            for g in (g0, g1, g2, g3):
                a = plsc.bitcast(g, jnp.int32) & jnp.int32(0x7FFFFFFF)
                mag = a if g is g0 else jnp.minimum(mag, a)
            pcnt = plsc.all_reduce_population_count(mag > 0)
            cnt = cnt + jnp.where(pcnt == 16, 1.0, 0.0)
            return (sums[0] + g0, sums[1] + g1, sums[2] + g2, sums[3] + g3), cnt

        def write_row(i, sums, cnt):
            div = jnp.where(cnt == 0.0, jnp.float32(1e-8), cnt)
            for c in range(4):
                ov[i, pl.ds(ecol + 16 * c, 16)] = sums[c] / div

        def reduce_pair(gbuf, p):
            def red(j, carry):
                sa, ca, sb, cb = carry
                for jj in (2 * j, 2 * j + 1):
                    sa, ca = row_step(gbuf, jj, sa, ca)
                    sb, cb = row_step(gbuf, HIST + jj, sb, cb)
                return (sa, ca, sb, cb)

            zero = jnp.zeros((16,), jnp.float32)
            z4 = (zero, zero, zero, zero)
            sa, ca, sb, cb = lax.fori_loop(0, HIST // 2, red, (z4, zero, z4, zero))
            write_row(2 * p, sa, ca)
            write_row(2 * p + 1, sb, cb)

        def chunk_body(chunk, _):
            base = wid * ROWS_PER_W + chunk * RCHUNK
            pltpu.sync_copy(
                x_hbm.at[pl.ds(base, RCHUNK), pl.ds(xcol0, xcols)], xv
            )

            for i in range(RCHUNK):
                if with_dense:
                    for c in range(DENSE // 16):
                        ov[i, pl.ds(16 * c, 16)] = xv[i, pl.ds(16 * c, 16)]
                half = (i % 2) * HIST
                for off in (0, 16, 32, HIST - 16):
                    idxv[i // 2, 0, pl.ds(half + off, 16)] = xv[
                        i, pl.ds(icol + off, 16)
                    ].astype(jnp.int32)

            def issue(k):
                return pltpu.async_copy(
                    w.at[idxv.at[k, 0]], gbufs[k % NBUF], sems[k % NBUF]
                )

            handles = {k: issue(k) for k in range(NBUF - 1)}
            for k in range(NPAIR):
                if k + NBUF - 1 < NPAIR:
                    handles[k + NBUF - 1] = issue(k + NBUF - 1)
                handles[k].wait()
                reduce_pair(gbufs[k % NBUF], k)

            pltpu.sync_copy(ov, out_hbm.at[pl.ds(base, RCHUNK), :])
            return ()

        lax.fori_loop(0, NCHUNK, chunk_body, ())

    mesh = plsc.VectorSubcoreMesh(core_axis_name="c", subcore_axis_name="s")
    return pl.kernel(
        body,
        out_type=jax.ShapeDtypeStruct((B, ocols), jnp.float32),
        mesh=mesh,
        compiler_params=pltpu.CompilerParams(
            needs_layout_passes=False, use_tc_tiling_on_sc=False
        ),
        scratch_types=[
            pltpu.VMEM((RCHUNK, xcols), jnp.float32),
            pltpu.VMEM((NPAIR, 1, 2 * HIST), jnp.int32),
            pltpu.VMEM((2 * HIST, EMB_DIM), jnp.float32),
            pltpu.VMEM((2 * HIST, EMB_DIM), jnp.float32),
            pltpu.VMEM((2 * HIST, EMB_DIM), jnp.float32),
            pltpu.VMEM((2 * HIST, EMB_DIM), jnp.float32),
            pltpu.VMEM((RCHUNK, ocols), jnp.float32),
            pltpu.SemaphoreType.DMA,
            pltpu.SemaphoreType.DMA,
            pltpu.SemaphoreType.DMA,
            pltpu.SemaphoreType.DMA,
        ],
        name=f"emb_pool_t{t}",
    )


_CALLS = [_make_body(t) for t in range(N_EMB)]


@jax.jit
def kernel(x, W0, W1, W2, W3):
    ws = (W0, W1, W2, W3)
    parts = [None] * N_EMB
    for t in (1, 2, 3, 0):
        parts[t] = _CALLS[t](x, ws[t])
    return jnp.concatenate(parts, axis=1)


# speedup vs baseline: 13.2333x; 1.0155x over previous
import jax
import jax.numpy as jnp
from jax import lax
from jax.experimental import pallas as pl
from jax.experimental.pallas import tpu as pltpu
from jax.experimental.pallas import tpu_sc as plsc

B = 4096
DENSE = 64
HIST = 50
N_EMB = 4
EMB_DIM = 64
XCOLS = DENSE + N_EMB * HIST

NC, NS = 2, 16
NW = NC * NS
ROWS_PER_W = B // NW
RCHUNK = 64
NCHUNK = ROWS_PER_W // RCHUNK
NPAIR = RCHUNK // 2
NBUF = 6


def _make_body(t):
    with_dense = t == 0
    ocols = DENSE + EMB_DIM if with_dense else EMB_DIM
    ecol = DENSE if with_dense else 0
    raw0 = 0 if with_dense else DENSE + t * HIST
    xcol0 = (raw0 // 8) * 8
    icol = (DENSE if with_dense else 0) + (raw0 - xcol0)
    xcols = ((icol + HIST + 7) // 8) * 8

    def body(
        x_hbm, w, out_hbm, xv, idxv,
        gb0, gb1, gb2, gb3, gb4, gb5, ov, s0, s1, s2, s3, s4, s5,
    ):
        gbufs = (gb0, gb1, gb2, gb3, gb4, gb5)
        sems = (s0, s1, s2, s3, s4, s5)
        wid = lax.axis_index("c") * NS + lax.axis_index("s")

        def row_step(gbuf, row, sums, cnt):
            g0 = gbuf[row, pl.ds(0, 16)]
            g1 = gbuf[row, pl.ds(16, 16)]
            g2 = gbuf[row, pl.ds(32, 16)]
            g3 = gbuf[row, pl.ds(48, 16)]
            mag = jnp.float32(0)
            for g in (g0, g1, g2, g3):
                a = plsc.bitcast(g, jnp.int32) & jnp.int32(0x7FFFFFFF)
                mag = a if g is g0 else jnp.minimum(mag, a)
            pcnt = plsc.all_reduce_population_count(mag > 0)
            cnt = cnt + jnp.where(pcnt == 16, 1.0, 0.0)
            return (sums[0] + g0, sums[1] + g1, sums[2] + g2, sums[3] + g3), cnt

        def write_row(i, sums, cnt):
            div = jnp.where(cnt == 0.0, jnp.float32(1e-8), cnt)
            for c in range(4):
                ov[i, pl.ds(ecol + 16 * c, 16)] = sums[c] / div

        def reduce_pair(gbuf, p):
            def red(j, carry):
                sa, ca, sb, cb = carry
                for jj in (2 * j, 2 * j + 1):
                    sa, ca = row_step(gbuf, jj, sa, ca)
                    sb, cb = row_step(gbuf, HIST + jj, sb, cb)
                return (sa, ca, sb, cb)

            zero = jnp.zeros((16,), jnp.float32)
            z4 = (zero, zero, zero, zero)
            sa, ca, sb, cb = lax.fori_loop(0, HIST // 2, red, (z4, zero, z4, zero))
            write_row(2 * p, sa, ca)
            write_row(2 * p + 1, sb, cb)

        def chunk_body(chunk, _):
            base = wid * ROWS_PER_W + chunk * RCHUNK
            pltpu.sync_copy(
                x_hbm.at[pl.ds(base, RCHUNK), pl.ds(xcol0, xcols)], xv
            )

            for i in range(RCHUNK):
                if with_dense:
                    for c in range(DENSE // 16):
                        ov[i, pl.ds(16 * c, 16)] = xv[i, pl.ds(16 * c, 16)]
                half = (i % 2) * HIST
                for off in (0, 16, 32, HIST - 16):
                    idxv[i // 2, 0, pl.ds(half + off, 16)] = xv[
                        i, pl.ds(icol + off, 16)
                    ].astype(jnp.int32)

            def issue(k):
                return pltpu.async_copy(
                    w.at[idxv.at[k, 0]], gbufs[k % NBUF], sems[k % NBUF]
                )

            handles = {k: issue(k) for k in range(NBUF - 1)}
            for k in range(NPAIR):
                if k + NBUF - 1 < NPAIR:
                    handles[k + NBUF - 1] = issue(k + NBUF - 1)
                handles[k].wait()
                reduce_pair(gbufs[k % NBUF], k)

            pltpu.sync_copy(ov, out_hbm.at[pl.ds(base, RCHUNK), :])
            return ()

        lax.fori_loop(0, NCHUNK, chunk_body, ())

    mesh = plsc.VectorSubcoreMesh(core_axis_name="c", subcore_axis_name="s")
    return pl.kernel(
        body,
        out_type=jax.ShapeDtypeStruct((B, ocols), jnp.float32),
        mesh=mesh,
        compiler_params=pltpu.CompilerParams(
            needs_layout_passes=False, use_tc_tiling_on_sc=False
        ),
        scratch_types=[
            pltpu.VMEM((RCHUNK, xcols), jnp.float32),
            pltpu.VMEM((NPAIR, 1, 2 * HIST), jnp.int32),
            pltpu.VMEM((2 * HIST, EMB_DIM), jnp.float32),
            pltpu.VMEM((2 * HIST, EMB_DIM), jnp.float32),
            pltpu.VMEM((2 * HIST, EMB_DIM), jnp.float32),
            pltpu.VMEM((2 * HIST, EMB_DIM), jnp.float32),
            pltpu.VMEM((2 * HIST, EMB_DIM), jnp.float32),
            pltpu.VMEM((2 * HIST, EMB_DIM), jnp.float32),
            pltpu.VMEM((RCHUNK, ocols), jnp.float32),
            pltpu.SemaphoreType.DMA,
            pltpu.SemaphoreType.DMA,
            pltpu.SemaphoreType.DMA,
            pltpu.SemaphoreType.DMA,
            pltpu.SemaphoreType.DMA,
            pltpu.SemaphoreType.DMA,
        ],
        name=f"emb_pool_t{t}",
    )


_CALLS = [_make_body(t) for t in range(N_EMB)]


@jax.jit
def kernel(x, W0, W1, W2, W3):
    ws = (W0, W1, W2, W3)
    parts = [None] * N_EMB
    for t in (1, 2, 3, 0):
        parts[t] = _CALLS[t](x, ws[t])
    return jnp.concatenate(parts, axis=1)
